# baseline (speedup 1.0000x reference)
import functools
import math

import jax
import jax.numpy as jnp
from jax import lax
from jax.experimental import pallas as pl
from jax.experimental.pallas import tpu as pltpu
from jax.experimental.pallas import tpu_sc as plsc

D_MODEL = 64
SCALE = math.sqrt(D_MODEL)

NC = 2
NS = 16
NW = NC * NS
L = 16

ROWS_PER_GATHER = 128
K_GATHERS = 8
G = ROWS_PER_GATHER * K_GATHERS


def _make_gather(B: int):
    b_per_w = B // NW
    n_groups = b_per_w // G
    assert b_per_w % G == 0

    mesh = plsc.VectorSubcoreMesh(core_axis_name="c", subcore_axis_name="s")

    @functools.partial(
        pl.kernel,
        out_type=jax.ShapeDtypeStruct((B, D_MODEL), jnp.float32),
        mesh=mesh,
        scratch_types=[
            pltpu.VMEM((K_GATHERS, ROWS_PER_GATHER), jnp.int32),
            pltpu.VMEM((G, D_MODEL), jnp.float32),
            pltpu.SemaphoreType.DMA,
        ],
        compiler_params=pltpu.CompilerParams(use_tc_tiling_on_sc=False),
    )
    def gather_scaled(lut_hbm, idx_hbm, out_hbm, idx_v, rows_v, sem):
        wid = lax.axis_index("s") * NC + lax.axis_index("c")
        base_row = wid * (b_per_w // ROWS_PER_GATHER)

        def group(g, carry):
            grow = base_row + g * K_GATHERS
            pltpu.sync_copy(idx_hbm.at[pl.ds(grow, K_GATHERS)], idx_v)
            copies = [
                pltpu.async_copy(
                    lut_hbm.at[idx_v.at[j]],
                    rows_v.at[pl.ds(j * ROWS_PER_GATHER, ROWS_PER_GATHER)],
                    sem,
                )
                for j in range(K_GATHERS)
            ]
            for c in copies:
                c.wait()

            def scale_row(r, carry2):
                for l in range(D_MODEL // L):
                    sl = pl.ds(l * L, L)
                    rows_v[r, sl] = rows_v[r, sl] * SCALE
                return carry2

            lax.fori_loop(0, G, scale_row, 0, unroll=2)
            pltpu.sync_copy(rows_v, out_hbm.at[pl.ds(grow * ROWS_PER_GATHER, G)])
            return carry

        lax.fori_loop(0, n_groups, group, 0)

    return gather_scaled


def kernel(x, lut):
    xs = x.shape
    B = xs[0] * xs[1]
    idx = x.reshape(B // ROWS_PER_GATHER, ROWS_PER_GATHER).astype(jnp.int32)
    out = _make_gather(B)(lut, idx)
    return out.reshape(xs[0], xs[1], D_MODEL)
